# Initial kernel scaffold; baseline (speedup 1.0000x reference)
import jax
import jax.numpy as jnp
from jax.experimental import pallas as pl


def kernel(logits, targets):
    raise NotImplementedError("write your pallas kernel here")


# speedup vs baseline: 16.9730x; 16.9730x over previous
import jax
import jax.numpy as jnp
import numpy as np
from jax import lax
from jax.experimental import pallas as pl
from jax.experimental.pallas import tpu as pltpu

LANES = 128
CHUNK = 512
KPOS_MAX = 128
BATCH = 512
TOPBIT = np.uint32(0x80000000)
LOWMASK = np.uint32(0x7FFFFFFF)


def _monokey(x, b):
    return jnp.where(b >= TOPBIT, ~b, b | TOPBIT)


def _unmono(ukey):
    b = jnp.where(ukey >= TOPBIT, ukey & LOWMASK, ~ukey)
    return lax.bitcast_convert_type(b, jnp.float32)


def _softplus(t):
    return jnp.maximum(t, 0.0) + jnp.log1p(jnp.exp(-jnp.abs(t)))


def _body(logits_ref, targets_ref, out_ref):
    rows = logits_ref.shape[0]
    nchunk = rows // CHUNK

    def _keys(i):
        x = logits_ref[pl.ds(i * CHUNK, CHUNK), :]
        t = targets_ref[pl.ds(i * CHUNK, CHUNK), :]
        b = lax.bitcast_convert_type(x, jnp.uint32)
        ukey = _monokey(x, b)
        kpos = jnp.where(t >= 1, ~ukey, np.uint32(0))
        kneg = jnp.where(t == 0, ukey, np.uint32(0))
        return kpos, kneg

    def count_chunk(i, carry):
        cpos, cneg = carry
        t = targets_ref[pl.ds(i * CHUNK, CHUNK), :]
        cpos = cpos + jnp.sum((t >= 1).astype(jnp.int32))
        cneg = cneg + jnp.sum((t == 0).astype(jnp.int32))
        return cpos, cneg

    cpos, cneg = lax.fori_loop(0, nchunk, count_chunk, (0, 0))
    k1 = jnp.minimum(cpos, KPOS_MAX)
    k2 = jnp.minimum(BATCH - k1, cneg)

    def bit_body(j, carry):
        ppos, pneg = carry
        bit = jnp.left_shift(np.uint32(1), (31 - j).astype(jnp.uint32))
        c1 = ppos | bit
        c2 = pneg | bit

        def cnt_chunk(i, acc):
            a1, a2 = acc
            kpos, kneg = _keys(i)
            a1 = a1 + jnp.sum((kpos >= c1).astype(jnp.int32))
            a2 = a2 + jnp.sum((kneg >= c2).astype(jnp.int32))
            return a1, a2

        n1, n2 = lax.fori_loop(0, nchunk, cnt_chunk, (0, 0))
        ppos = jnp.where(n1 >= k1, c1, ppos)
        pneg = jnp.where(n2 >= k2, c2, pneg)
        return ppos, pneg

    tpos, tneg = lax.fori_loop(0, 32, bit_body, (jnp.uint32(0), jnp.uint32(0)))

    def fin_chunk(i, carry):
        s1, s2, g1, g2 = carry
        kpos, kneg = _keys(i)
        sel1 = kpos > tpos
        sel2 = kneg > tneg
        x1 = _unmono(~kpos)
        x2 = _unmono(kneg)
        l1 = jnp.minimum(_softplus(-x1), 100.0)
        l2 = jnp.minimum(_softplus(x2), 100.0)
        s1 = s1 + jnp.sum(jnp.where(sel1, l1, 0.0))
        s2 = s2 + jnp.sum(jnp.where(sel2, l2, 0.0))
        g1 = g1 + jnp.sum(sel1.astype(jnp.int32))
        g2 = g2 + jnp.sum(sel2.astype(jnp.int32))
        return s1, s2, g1, g2

    s1, s2, g1, g2 = lax.fori_loop(0, nchunk, fin_chunk, (0.0, 0.0, 0, 0))

    tl1 = jnp.minimum(_softplus(-_unmono(~tpos)), 100.0)
    tl2 = jnp.minimum(_softplus(_unmono(tneg)), 100.0)
    s1 = s1 + jnp.where(k1 > g1, (k1 - g1).astype(jnp.float32) * tl1, 0.0)
    s2 = s2 + jnp.where(k2 > g2, (k2 - g2).astype(jnp.float32) * tl2, 0.0)
    out_ref[...] = jnp.broadcast_to((s1 + s2) / float(BATCH), (1, 1))


def kernel(logits, targets):
    n = logits.shape[0]
    rows = n // LANES
    l2 = logits.reshape(rows, LANES)
    t2 = targets.reshape(rows, LANES)
    out = pl.pallas_call(
        _body,
        out_shape=jax.ShapeDtypeStruct((1, 1), jnp.float32),
        in_specs=[
            pl.BlockSpec(memory_space=pltpu.VMEM),
            pl.BlockSpec(memory_space=pltpu.VMEM),
        ],
        out_specs=pl.BlockSpec(memory_space=pltpu.VMEM),
    )(l2, t2)
    return out[0, 0]
